# baseline (speedup 1.0000x reference)
import functools

import jax
import jax.numpy as jnp
from jax import lax
from jax.experimental import pallas as pl
from jax.experimental.pallas import tpu as pltpu
from jax.experimental.pallas import tpu_sc as plsc

_NC, _NS, _L = 2, 16, 16
_NW = _NC * _NS
_N = 320000
_G = 2000
_D = 128
_E = 16
_P = _N // _NW
_CHUNK = 200
_NCH = _P // _CHUNK
_NB = 3
_SUB = 100
_NSUB = _CHUNK // _SUB
_IDR = _P // _SUB
_GP = 2048
_RPT = _GP // _NS


def _sc_body(emb, ids3, out, rows, ids_v, acc_sh, sem_g):
    cid = lax.axis_index("c")
    sid = lax.axis_index("s")
    wid = cid * _NS + sid
    base = wid * _P

    def issue(c, b):
        pltpu.async_copy(emb.at[pl.ds(base + c * _CHUNK, _CHUNK)], rows[b], sem_g[b])

    def drain(c, b):
        pltpu.make_async_copy(emb.at[pl.ds(base + c * _CHUNK, _CHUNK)], rows[b],
                              sem_g[b]).wait()

    def scatter(c, b):
        for s in range(_NSUB):
            pltpu.sync_copy(rows[b].at[pl.ds(s * _SUB, _SUB)],
                            acc_sh.at[ids_v.at[c * _NSUB + s]], add=True)

    @pl.loop(0, _RPT)
    def _(r):
        for f in range(_D // _L):
            rows[0][r, pl.ds(f * _L, _L)] = jnp.zeros((_L,), jnp.float32)

    pltpu.sync_copy(rows[0].at[pl.ds(0, _RPT)], acc_sh.at[pl.ds(sid * _RPT, _RPT)])

    pltpu.sync_copy(ids3.at[wid], ids_v)
    plsc.subcore_barrier()

    for b in range(_NB):
        issue(b, b)

    @pl.loop(0, _NCH - _NCH % _NB, step=_NB)
    def _(i):
        for j in range(_NB):
            c = i + j
            drain(c, j)
            scatter(c, j)

            @pl.when(c + _NB < _NCH)
            def _():
                issue(c + _NB, j)

    for c in range(_NCH - _NCH % _NB, _NCH):
        b = c % _NB
        drain(c, b)
        scatter(c, b)

    plsc.subcore_barrier()
    row0 = cid * _GP + sid * _RPT
    pltpu.sync_copy(acc_sh.at[pl.ds(sid * _RPT, _RPT)], out.at[pl.ds(row0, _RPT)])


_sc_segsum = pl.kernel(
    _sc_body,
    out_type=jax.ShapeDtypeStruct((_NC * _GP, _D), jnp.float32),
    mesh=plsc.VectorSubcoreMesh(core_axis_name="c", subcore_axis_name="s"),
    scratch_types=[
        [pltpu.VMEM((_CHUNK, _D), jnp.float32) for _ in range(_NB)],
        pltpu.VMEM((_IDR, _SUB), jnp.int32),
        pltpu.VMEM_SHARED((_GP, _D), jnp.float32),
        [pltpu.SemaphoreType.DMA for _ in range(_NB)],
    ],
)


def _tc_final(p_ref, wmu_ref, wvar_ref, bmu_ref, bvar_ref, mu_ref, var_ref):
    s = p_ref[0:_G, :] + p_ref[_GP:_GP + _G, :]
    dims = (((1,), (1,)), ((), ()))
    mu_ref[...] = (
        lax.dot_general(s, wmu_ref[...], dims, preferred_element_type=jnp.float32)
        + bmu_ref[...]
    )
    pre = (
        lax.dot_general(s, wvar_ref[...], dims, preferred_element_type=jnp.float32)
        + bvar_ref[...]
    )
    var_ref[...] = jax.nn.softplus(pre) + 1e-8


_tc_call = pl.pallas_call(
    _tc_final,
    out_shape=[
        jax.ShapeDtypeStruct((_G, _E), jnp.float32),
        jax.ShapeDtypeStruct((_G, _E), jnp.float32),
    ],
)


@jax.jit
def kernel(node_embeddings, batch, W, b):
    partials = _sc_segsum(node_embeddings, batch.reshape(_NW, _IDR, _SUB))
    w_mu = W[0::2]
    w_var = W[1::2]
    b_mu = b[0::2].reshape(1, _E)
    b_var = b[1::2].reshape(1, _E)
    mu, var = _tc_call(partials, w_mu, w_var, b_mu, b_var)
    return mu[:, :, None], var[:, :, None]
